# baseline (speedup 1.0000x reference)
import functools

import jax
import jax.numpy as jnp
from jax import lax
from jax.experimental import pallas as pl
from jax.experimental.pallas import tpu as pltpu
from jax.experimental.pallas import tpu_sc as plsc

SC_COLS = 0


def _tc_body(ga_ref, gb_ref, gc_ref, guide_ref, out_ref, *,
             scale, D, C, Wg, W, Wfull, wblk0):
    half = scale // 2
    woff = ((pl.program_id(2) + wblk0) * W).astype(jnp.float32)
    wpos = jax.lax.broadcasted_iota(jnp.int32, (1, W), 1).astype(jnp.float32)
    gx = jnp.clip((wpos + woff + 0.5) * (Wg / Wfull) - 0.5, 0.0, Wg - 1.0)
    xg = jax.lax.broadcasted_iota(jnp.int32, (Wg, 1), 0).astype(jnp.float32)
    Bx = jnp.maximum(0.0, 1.0 - jnp.abs(gx - xg))

    ga = ga_ref[0, 0].reshape(C * D, Wg)
    gb = gb_ref[0, 0].reshape(C * D, Wg)
    gc = gc_ref[0, 0].reshape(C * D, Wg)
    A = jnp.dot(ga, Bx, preferred_element_type=jnp.float32)
    B = jnp.dot(gb, Bx, preferred_element_type=jnp.float32)
    Cc = jnp.dot(gc, Bx, preferred_element_type=jnp.float32)

    jrow = jax.lax.broadcasted_iota(jnp.int32, (half, 1), 0).astype(jnp.float32)
    for h in range(2):
        rows = guide_ref[0, 0, h * half:(h + 1) * half, :]
        gz = jnp.clip(rows * D - 0.5, 0.0, D - 1.0)
        if h == 0:
            base, diff = A, B - A
            wy = (jrow + 0.5) / scale + 0.5
        else:
            base, diff = B, Cc - B
            wy = (jrow + 0.5) / scale
        base = base.reshape(C, D, W)
        diff = diff.reshape(C, D, W)
        u = [jnp.maximum(0.0, 1.0 - jnp.abs(gz - d)) for d in range(D)]
        v = [u[d] * wy for d in range(D)]
        for c in range(C):
            acc = u[0] * base[c, 0][None, :] + v[0] * diff[c, 0][None, :]
            for d in range(1, D):
                acc = acc + u[d] * base[c, d][None, :]
                acc = acc + v[d] * diff[c, d][None, :]
            out_ref[0, c, h * half:(h + 1) * half, :] = acc


def _tc_slice(gridT, guidemap, sc_cols):
    Bn, Hg, C, D, Wg = gridT.shape
    H, Wfull = guidemap.shape[2], guidemap.shape[3]
    scale = H // Hg
    WB = 128
    wblk0 = sc_cols // WB
    nwb = (Wfull - sc_cols) // WB
    body = functools.partial(_tc_body, scale=scale, D=D, C=C, Wg=Wg, W=WB,
                             Wfull=Wfull, wblk0=wblk0)

    def gmap(off):
        def imap(b, k, j):
            return (b, jnp.clip(k + off, 0, Hg - 1), 0, 0, 0)
        return imap

    return pl.pallas_call(
        body,
        grid=(Bn, Hg, nwb),
        in_specs=[
            pl.BlockSpec((1, 1, C, D, Wg), gmap(-1)),
            pl.BlockSpec((1, 1, C, D, Wg), gmap(0)),
            pl.BlockSpec((1, 1, C, D, Wg), gmap(1)),
            pl.BlockSpec((1, 1, scale, WB),
                         lambda b, k, j: (b, 0, k, j + wblk0)),
        ],
        out_specs=pl.BlockSpec((1, C, scale, WB),
                               lambda b, k, j: (b, 0, k, j + wblk0)),
        out_shape=jax.ShapeDtypeStruct((Bn, C, H, Wfull), jnp.float32),
        compiler_params=pltpu.CompilerParams(
            dimension_semantics=("parallel", "arbitrary", "arbitrary"),
        ),
    )(gridT, gridT, gridT, guidemap)


def _sc_slice(grid3, guidemap, sc_cols, Hg, C, D, Wg):
    Bn = grid3.shape[0]
    CD = C * D
    H, Wfull = guidemap.shape[2], guidemap.shape[3]
    scale = H // Hg
    CW = 128
    NCH = CW // 16
    ncs = sc_cols // CW
    nrg = 4 // ncs
    nb = Hg // nrg
    mesh = plsc.VectorSubcoreMesh(core_axis_name="c", subcore_axis_name="s")

    @functools.partial(
        pl.kernel, mesh=mesh,
        out_type=jax.ShapeDtypeStruct((Bn, C, H, sc_cols), jnp.float32),
        compiler_params=pltpu.CompilerParams(needs_layout_passes=False),
        scratch_types=[
            pltpu.VMEM((Hg * CD * Wg,), jnp.float32),
            pltpu.VMEM((3 * CD * CW,), jnp.float32),
            pltpu.VMEM((2, scale, CW), jnp.float32),
            pltpu.VMEM((C, scale // 2, CW), jnp.float32),
            pltpu.VMEM((C, scale // 2, CW), jnp.float32),
            pltpu.SemaphoreType.DMA,
            pltpu.SemaphoreType.DMA,
            pltpu.SemaphoreType.DMA,
        ],
    )
    def sck(grid_hbm, guide_hbm, out_hbm, graw, rbuf, gbuf, obuf0, obuf1,
            gsem, osem0, osem1):
        wid = lax.axis_index("s") * 2 + lax.axis_index("c")
        b = wid // 4
        t = wid % 4
        col0 = lax.rem(t, ncs) * CW
        k0 = (t // ncs) * nb
        lane = lax.iota(jnp.int32, 16)
        half = scale // 2

        def guide_src(k):
            return guide_hbm.at[b, 0, pl.ds(k * scale, scale),
                                pl.ds(col0, CW)]

        def out_dst(k, h):
            return out_hbm.at[b, :, pl.ds(k * scale + h * half, half),
                              pl.ds(col0, CW)]

        pltpu.async_copy(guide_src(k0), gbuf.at[lax.rem(k0, 2)], gsem)
        pltpu.sync_copy(grid_hbm.at[b], graw)

        def upsample(y, slot, _carry):
            UNR = 8

            def ch_body(j, _):
                w = col0 + j * 16 + lane
                gx = jnp.clip((w.astype(jnp.float32) + 0.5) * (Wg / Wfull)
                              - 0.5, 0.0, Wg - 1.0)
                ix0 = jnp.minimum(gx.astype(jnp.int32), Wg - 2)
                wx = gx - ix0.astype(jnp.float32)
                gidx = (y * CD) * Wg + ix0
                rb0 = (slot * CD) * CW + j * 16

                def cd_body(cd, _):
                    pairs = [(plsc.load_gather(graw, [gidx + (cd + u) * Wg]),
                              plsc.load_gather(graw, [gidx + (cd + u) * Wg + 1]))
                             for u in range(UNR)]
                    for u, (g0, g1) in enumerate(pairs):
                        rbuf[pl.ds(rb0 + (cd + u) * CW, 16)] = \
                            g0 + wx * (g1 - g0)
                    return 0

                lax.fori_loop(0, CD // UNR, lambda i, c: cd_body(i * UNR, c), 0)
                return 0

            lax.fori_loop(0, NCH, ch_body, 0)
            return 0

        y0 = jnp.maximum(k0 - 1, 0)
        upsample(y0, lax.rem(y0, 3), 0)
        upsample(k0, lax.rem(k0, 3), 0)

        def band_body(k, _):
            kp = lax.rem(k, 2)
            pltpu.make_async_copy(guide_src(k), gbuf.at[kp], gsem).wait()

            @pl.when(k < k0 + nb - 1)
            def _():
                pltpu.async_copy(guide_src(k + 1), gbuf.at[1 - kp], gsem)

            ynext = jnp.minimum(k + 1, Hg - 1)
            upsample(ynext, lax.rem(ynext, 3), 0)
            ya = jnp.maximum(k - 1, 0)
            sa = lax.rem(ya, 3)
            sb = lax.rem(k, 3)
            sc_ = lax.rem(ynext, 3)

            for h, (obuf, osem) in enumerate(((obuf0, osem0), (obuf1, osem1))):
                s0 = sa if h == 0 else sb
                s1 = sb if h == 0 else sc_
                s0base = s0 * (CD * CW)
                s1base = s1 * (CD * CW)

                @pl.when(k > k0)
                def _():
                    pltpu.make_async_copy(obuf, out_dst(k - 1, h), osem).wait()

                def row_body(rr, _):
                    wy = ((rr.astype(jnp.float32) + 0.5) * (1.0 / scale)
                          + (0.5 if h == 0 else 0.0))

                    def ch_body(j, _):
                        wl = j * 16 + lane
                        g = gbuf[kp, h * half + rr, pl.ds(j * 16, 16)]
                        gz = jnp.clip(g * D - 0.5, 0.0, D - 1.0)
                        iz0 = jnp.minimum(gz.astype(jnp.int32), D - 2)
                        wz = gz - iz0.astype(jnp.float32)
                        zoff = iz0 * CW + wl
                        f0 = zoff + s0base
                        f1 = zoff + s1base
                        loads = [(plsc.load_gather(rbuf, [f0 + (c * D * CW)]),
                                  plsc.load_gather(rbuf, [f1 + (c * D * CW)]),
                                  plsc.load_gather(rbuf, [f0 + (c * D * CW + CW)]),
                                  plsc.load_gather(rbuf, [f1 + (c * D * CW + CW)]))
                                 for c in range(C)]
                        for c, (a0, b0, a1, b1) in enumerate(loads):
                            r0 = a0 + wy * (b0 - a0)
                            r1 = a1 + wy * (b1 - a1)
                            obuf[c, rr, pl.ds(j * 16, 16)] = r0 + wz * (r1 - r0)
                        return 0

                    lax.fori_loop(0, NCH, ch_body, 0)
                    return 0

                lax.fori_loop(0, half, row_body, 0)
                pltpu.async_copy(obuf, out_dst(k, h), osem)
            return 0

        lax.fori_loop(k0, k0 + nb, band_body, 0)
        pltpu.make_async_copy(obuf0, out_dst(k0 + nb - 1, 0), osem0).wait()
        pltpu.make_async_copy(obuf1, out_dst(k0 + nb - 1, 1), osem1).wait()

    return sck(grid3, guidemap)


def kernel(bilateral_grid, guidemap):
    Bn, C, D, Hg, Wg = bilateral_grid.shape
    H, W = guidemap.shape[2], guidemap.shape[3]
    gridT = jnp.transpose(bilateral_grid, (0, 3, 1, 2, 4))

    if SC_COLS == 0:
        return _tc_slice(gridT, guidemap, 0)
    grid3 = gridT.reshape(Bn, Hg * C * D * Wg)
    sc_part = _sc_slice(grid3, guidemap, SC_COLS, Hg, C, D, Wg)
    if SC_COLS == W:
        return sc_part
    out = _tc_slice(gridT, guidemap, SC_COLS)
    return jax.lax.dynamic_update_slice(out, sc_part, (0, 0, 0, 0))


# speedup vs baseline: 2.0651x; 2.0651x over previous
import functools

import jax
import jax.numpy as jnp
from jax import lax
from jax.experimental import pallas as pl
from jax.experimental.pallas import tpu as pltpu
from jax.experimental.pallas import tpu_sc as plsc

SC_ROWS = 192


def _tc_body(ga_ref, gb_ref, gc_ref, guide_ref, out_ref, *,
             scale, D, C, Wg, W):
    half = scale // 2
    wpos = jax.lax.broadcasted_iota(jnp.int32, (1, W), 1).astype(jnp.float32)
    gx = jnp.clip((wpos + 0.5) * (Wg / W) - 0.5, 0.0, Wg - 1.0)
    xg = jax.lax.broadcasted_iota(jnp.int32, (Wg, 1), 0).astype(jnp.float32)
    Bx = jnp.maximum(0.0, 1.0 - jnp.abs(gx - xg))

    ga = ga_ref[0, 0].reshape(C * D, Wg)
    gb = gb_ref[0, 0].reshape(C * D, Wg)
    gc = gc_ref[0, 0].reshape(C * D, Wg)
    A = jnp.dot(ga, Bx, preferred_element_type=jnp.float32)
    B = jnp.dot(gb, Bx, preferred_element_type=jnp.float32)
    Cc = jnp.dot(gc, Bx, preferred_element_type=jnp.float32)

    jrow = jax.lax.broadcasted_iota(jnp.int32, (half, 1), 0).astype(jnp.float32)
    for h in range(2):
        rows = guide_ref[0, 0, h * half:(h + 1) * half, :]
        gz = jnp.clip(rows * D - 0.5, 0.0, D - 1.0)
        if h == 0:
            base, diff = A, B - A
            wy = (jrow + 0.5) / scale + 0.5
        else:
            base, diff = B, Cc - B
            wy = (jrow + 0.5) / scale
        base = base.reshape(C, D, W)
        diff = diff.reshape(C, D, W)
        u = [jnp.maximum(0.0, 1.0 - jnp.abs(gz - d)) for d in range(D)]
        v = [u[d] * wy for d in range(D)]
        for c in range(C):
            acc = u[0] * base[c, 0][None, :] + v[0] * diff[c, 0][None, :]
            for d in range(1, D):
                acc = acc + u[d] * base[c, d][None, :]
                acc = acc + v[d] * diff[c, d][None, :]
            out_ref[0, c, h * half:(h + 1) * half, :] = acc


def _tc_slice(gridT, guidemap, sc_rows):
    Bn, Hg, C, D, Wg = gridT.shape
    H, W = guidemap.shape[2], guidemap.shape[3]
    scale = H // Hg
    kofs = sc_rows // scale
    body = functools.partial(_tc_body, scale=scale, D=D, C=C, Wg=Wg, W=W)

    def gmap(off):
        def imap(b, k):
            return (b, jnp.clip(k + kofs + off, 0, Hg - 1), 0, 0, 0)
        return imap

    return pl.pallas_call(
        body,
        grid=(Bn, Hg - kofs),
        in_specs=[
            pl.BlockSpec((1, 1, C, D, Wg), gmap(-1)),
            pl.BlockSpec((1, 1, C, D, Wg), gmap(0)),
            pl.BlockSpec((1, 1, C, D, Wg), gmap(1)),
            pl.BlockSpec((1, 1, scale, W), lambda b, k: (b, 0, k + kofs, 0)),
        ],
        out_specs=pl.BlockSpec((1, C, scale, W),
                               lambda b, k: (b, 0, k + kofs, 0)),
        out_shape=jax.ShapeDtypeStruct((Bn, C, H, W), jnp.float32),
        compiler_params=pltpu.CompilerParams(
            dimension_semantics=("parallel", "arbitrary"),
        ),
    )(gridT, gridT, gridT, guidemap)


def _sc_slice(grid3, guidemap, sc_rows, Hg, C, D, Wg):
    Bn = grid3.shape[0]
    CD = C * D
    H, Wfull = guidemap.shape[2], guidemap.shape[3]
    scale = H // Hg
    CW = 128
    NCH = CW // 16
    ncs = Wfull // CW
    nb = sc_rows // scale
    mesh = plsc.VectorSubcoreMesh(core_axis_name="c", subcore_axis_name="s")

    @functools.partial(
        pl.kernel, mesh=mesh,
        out_type=jax.ShapeDtypeStruct((Bn, C, sc_rows, Wfull), jnp.float32),
        compiler_params=pltpu.CompilerParams(needs_layout_passes=False),
        scratch_types=[
            pltpu.VMEM((Hg * CD * Wg,), jnp.float32),
            pltpu.VMEM((3 * CD * CW,), jnp.float32),
            pltpu.VMEM((2, scale, CW), jnp.float32),
            pltpu.VMEM((C, scale // 2, CW), jnp.float32),
            pltpu.VMEM((C, scale // 2, CW), jnp.float32),
            pltpu.SemaphoreType.DMA,
            pltpu.SemaphoreType.DMA,
            pltpu.SemaphoreType.DMA,
        ],
    )
    def sck(grid_hbm, guide_hbm, out_hbm, graw, rbuf, gbuf, obuf0, obuf1,
            gsem, osem0, osem1):
        wid = lax.axis_index("s") * 2 + lax.axis_index("c")
        b = wid // ncs
        col0 = lax.rem(wid, ncs) * CW
        k0 = 0
        lane = lax.iota(jnp.int32, 16)
        half = scale // 2

        def guide_src(k):
            return guide_hbm.at[b, 0, pl.ds(k * scale, scale),
                                pl.ds(col0, CW)]

        def out_dst(k, h):
            return out_hbm.at[b, :, pl.ds(k * scale + h * half, half),
                              pl.ds(col0, CW)]

        pltpu.async_copy(guide_src(k0), gbuf.at[k0 % 2], gsem)
        pltpu.sync_copy(grid_hbm.at[b], graw)

        def upsample(y, slot, _carry):
            UNR = 8

            def ch_body(j, _):
                w = col0 + j * 16 + lane
                gx = jnp.clip((w.astype(jnp.float32) + 0.5) * (Wg / Wfull)
                              - 0.5, 0.0, Wg - 1.0)
                ix0 = jnp.minimum(gx.astype(jnp.int32), Wg - 2)
                wx = gx - ix0.astype(jnp.float32)
                gidx = (y * CD) * Wg + ix0
                rb0 = (slot * CD) * CW + j * 16

                def cd_body(cd, _):
                    pairs = [(plsc.load_gather(graw, [gidx + (cd + u) * Wg]),
                              plsc.load_gather(graw, [gidx + (cd + u) * Wg + 1]))
                             for u in range(UNR)]
                    for u, (g0, g1) in enumerate(pairs):
                        rbuf[pl.ds(rb0 + (cd + u) * CW, 16)] = \
                            g0 + wx * (g1 - g0)
                    return 0

                lax.fori_loop(0, CD // UNR, lambda i, c: cd_body(i * UNR, c), 0)
                return 0

            lax.fori_loop(0, NCH, ch_body, 0)
            return 0

        if k0 > 0:
            upsample(k0 - 1, (k0 - 1) % 3, 0)
        upsample(k0, k0 % 3, 0)

        def band_body(k, _):
            kp = lax.rem(k, 2)
            pltpu.make_async_copy(guide_src(k), gbuf.at[kp], gsem).wait()

            @pl.when(k < k0 + nb - 1)
            def _():
                pltpu.async_copy(guide_src(k + 1), gbuf.at[1 - kp], gsem)

            ynext = jnp.minimum(k + 1, Hg - 1)
            upsample(ynext, lax.rem(ynext, 3), 0)
            ya = jnp.maximum(k - 1, 0)
            sa = lax.rem(ya, 3)
            sb = lax.rem(k, 3)
            sc_ = lax.rem(ynext, 3)

            for h, (obuf, osem) in enumerate(((obuf0, osem0), (obuf1, osem1))):
                s0 = sa if h == 0 else sb
                s1 = sb if h == 0 else sc_
                s0base = s0 * (CD * CW)
                s1base = s1 * (CD * CW)

                @pl.when(k > k0)
                def _():
                    pltpu.make_async_copy(obuf, out_dst(k - 1, h), osem).wait()

                def row_body(rr, _):
                    wy = ((rr.astype(jnp.float32) + 0.5) * (1.0 / scale)
                          + (0.5 if h == 0 else 0.0))

                    def ch_body(j, _):
                        wl = j * 16 + lane
                        g = gbuf[kp, h * half + rr, pl.ds(j * 16, 16)]
                        gz = jnp.clip(g * D - 0.5, 0.0, D - 1.0)
                        iz0 = jnp.minimum(gz.astype(jnp.int32), D - 2)
                        wz = gz - iz0.astype(jnp.float32)
                        zoff = iz0 * CW + wl
                        f0 = zoff + s0base
                        f1 = zoff + s1base
                        loads = [(plsc.load_gather(rbuf, [f0 + (c * D * CW)]),
                                  plsc.load_gather(rbuf, [f1 + (c * D * CW)]),
                                  plsc.load_gather(rbuf, [f0 + (c * D * CW + CW)]),
                                  plsc.load_gather(rbuf, [f1 + (c * D * CW + CW)]))
                                 for c in range(C)]
                        for c, (a0, b0, a1, b1) in enumerate(loads):
                            r0 = a0 + wy * (b0 - a0)
                            r1 = a1 + wy * (b1 - a1)
                            obuf[c, rr, pl.ds(j * 16, 16)] = r0 + wz * (r1 - r0)
                        return 0

                    lax.fori_loop(0, NCH, ch_body, 0)
                    return 0

                lax.fori_loop(0, half, row_body, 0)
                pltpu.async_copy(obuf, out_dst(k, h), osem)
            return 0

        lax.fori_loop(k0, k0 + nb, band_body, 0)
        pltpu.make_async_copy(obuf0, out_dst(k0 + nb - 1, 0), osem0).wait()
        pltpu.make_async_copy(obuf1, out_dst(k0 + nb - 1, 1), osem1).wait()

    return sck(grid3, guidemap)


def kernel(bilateral_grid, guidemap):
    Bn, C, D, Hg, Wg = bilateral_grid.shape
    H, W = guidemap.shape[2], guidemap.shape[3]
    gridT = jnp.transpose(bilateral_grid, (0, 3, 1, 2, 4))

    if SC_ROWS == 0:
        return _tc_slice(gridT, guidemap, 0)
    grid3 = gridT.reshape(Bn, Hg * C * D * Wg)
    sc_part = _sc_slice(grid3, guidemap, SC_ROWS, Hg, C, D, Wg)
    if SC_ROWS == H:
        return sc_part
    out = _tc_slice(gridT, guidemap, SC_ROWS)
    return jax.lax.dynamic_update_slice(out, sc_part, (0, 0, 0, 0))


# speedup vs baseline: 2.1687x; 1.0502x over previous
import functools

import jax
import jax.numpy as jnp
from jax import lax
from jax.experimental import pallas as pl
from jax.experimental.pallas import tpu as pltpu
from jax.experimental.pallas import tpu_sc as plsc

SC_ROWS = 192


def _tc_body(ga_ref, gb_ref, gc_ref, gd_ref, guide_ref, out_ref, *,
             scale, D, C, Wg, W):
    half = scale // 2
    wpos = jax.lax.broadcasted_iota(jnp.int32, (1, W), 1).astype(jnp.float32)
    gx = jnp.clip((wpos + 0.5) * (Wg / W) - 0.5, 0.0, Wg - 1.0)
    xg = jax.lax.broadcasted_iota(jnp.int32, (Wg, 1), 0).astype(jnp.float32)
    Bx = jnp.maximum(0.0, 1.0 - jnp.abs(gx - xg))

    ga = ga_ref[0, 0].reshape(C * D, Wg)
    gb = gb_ref[0, 0].reshape(C * D, Wg)
    gc = gc_ref[0, 0].reshape(C * D, Wg)
    gd = gd_ref[0, 0].reshape(C * D, Wg)
    A = jnp.dot(ga, Bx, preferred_element_type=jnp.float32)
    B = jnp.dot(gb, Bx, preferred_element_type=jnp.float32)
    Cc = jnp.dot(gc, Bx, preferred_element_type=jnp.float32)
    Dd = jnp.dot(gd, Bx, preferred_element_type=jnp.float32)

    sections = [(0, half, A, B - A, 0.5),
                (half, 3 * half, B, Cc - B, 0.0),
                (3 * half, 4 * half, Cc, Dd - Cc, 0.0)]
    for r0, r1, base, diff, w0 in sections:
        n = r1 - r0
        jrow = jax.lax.broadcasted_iota(jnp.int32, (n, 1), 0)
        wy = (jrow.astype(jnp.float32) + 0.5) / scale + w0
        rows = guide_ref[0, 0, r0:r1, :]
        gz = jnp.clip(rows * D - 0.5, 0.0, D - 1.0)
        base = base.reshape(C, D, W)
        diff = diff.reshape(C, D, W)
        u = [jnp.maximum(0.0, 1.0 - jnp.abs(gz - d)) for d in range(D)]
        v = [u[d] * wy for d in range(D)]
        for c in range(C):
            acc = u[0] * base[c, 0][None, :] + v[0] * diff[c, 0][None, :]
            for d in range(1, D):
                acc = acc + u[d] * base[c, d][None, :]
                acc = acc + v[d] * diff[c, d][None, :]
            out_ref[0, c, r0:r1, :] = acc


def _tc_slice(gridT, guidemap, sc_rows):
    Bn, Hg, C, D, Wg = gridT.shape
    H, W = guidemap.shape[2], guidemap.shape[3]
    scale = H // Hg
    kofs = sc_rows // scale
    body = functools.partial(_tc_body, scale=scale, D=D, C=C, Wg=Wg, W=W)

    def gmap(off):
        def imap(b, p):
            return (b, jnp.clip(2 * p + kofs + off, 0, Hg - 1), 0, 0, 0)
        return imap

    return pl.pallas_call(
        body,
        grid=(Bn, (Hg - kofs) // 2),
        in_specs=[
            pl.BlockSpec((1, 1, C, D, Wg), gmap(-1)),
            pl.BlockSpec((1, 1, C, D, Wg), gmap(0)),
            pl.BlockSpec((1, 1, C, D, Wg), gmap(1)),
            pl.BlockSpec((1, 1, C, D, Wg), gmap(2)),
            pl.BlockSpec((1, 1, 2 * scale, W),
                         lambda b, p: (b, 0, p + kofs // 2, 0)),
        ],
        out_specs=pl.BlockSpec((1, C, 2 * scale, W),
                               lambda b, p: (b, 0, p + kofs // 2, 0)),
        out_shape=jax.ShapeDtypeStruct((Bn, C, H, W), jnp.float32),
        compiler_params=pltpu.CompilerParams(
            dimension_semantics=("parallel", "arbitrary"),
        ),
    )(gridT, gridT, gridT, gridT, guidemap)


def _sc_slice(grid3, guidemap, sc_rows, Hg, C, D, Wg):
    Bn = grid3.shape[0]
    CD = C * D
    H, Wfull = guidemap.shape[2], guidemap.shape[3]
    scale = H // Hg
    CW = 128
    NCH = CW // 16
    ncs = Wfull // CW
    nb = sc_rows // scale
    mesh = plsc.VectorSubcoreMesh(core_axis_name="c", subcore_axis_name="s")

    @functools.partial(
        pl.kernel, mesh=mesh,
        out_type=jax.ShapeDtypeStruct((Bn, C, sc_rows, Wfull), jnp.float32),
        compiler_params=pltpu.CompilerParams(needs_layout_passes=False),
        scratch_types=[
            pltpu.VMEM((Hg * CD * Wg,), jnp.float32),
            pltpu.VMEM((3 * CD * CW,), jnp.float32),
            pltpu.VMEM((2, scale, CW), jnp.float32),
            pltpu.VMEM((C, scale // 2, CW), jnp.float32),
            pltpu.VMEM((C, scale // 2, CW), jnp.float32),
            pltpu.SemaphoreType.DMA,
            pltpu.SemaphoreType.DMA,
            pltpu.SemaphoreType.DMA,
        ],
    )
    def sck(grid_hbm, guide_hbm, out_hbm, graw, rbuf, gbuf, obuf0, obuf1,
            gsem, osem0, osem1):
        wid = lax.axis_index("s") * 2 + lax.axis_index("c")
        b = wid // ncs
        col0 = lax.rem(wid, ncs) * CW
        k0 = 0
        lane = lax.iota(jnp.int32, 16)
        half = scale // 2

        def guide_src(k):
            return guide_hbm.at[b, 0, pl.ds(k * scale, scale),
                                pl.ds(col0, CW)]

        def out_dst(k, h):
            return out_hbm.at[b, :, pl.ds(k * scale + h * half, half),
                              pl.ds(col0, CW)]

        pltpu.async_copy(guide_src(k0), gbuf.at[k0 % 2], gsem)
        pltpu.sync_copy(grid_hbm.at[b], graw)

        def upsample(y, slot, _carry):
            UNR = 8

            def ch_body(j, _):
                w = col0 + j * 16 + lane
                gx = jnp.clip((w.astype(jnp.float32) + 0.5) * (Wg / Wfull)
                              - 0.5, 0.0, Wg - 1.0)
                ix0 = jnp.minimum(gx.astype(jnp.int32), Wg - 2)
                wx = gx - ix0.astype(jnp.float32)
                gidx = (y * CD) * Wg + ix0
                rb0 = (slot * CD) * CW + j * 16

                def cd_body(cd, _):
                    pairs = [(plsc.load_gather(graw, [gidx + (cd + u) * Wg]),
                              plsc.load_gather(graw, [gidx + (cd + u) * Wg + 1]))
                             for u in range(UNR)]
                    for u, (g0, g1) in enumerate(pairs):
                        rbuf[pl.ds(rb0 + (cd + u) * CW, 16)] = \
                            g0 + wx * (g1 - g0)
                    return 0

                lax.fori_loop(0, CD // UNR, lambda i, c: cd_body(i * UNR, c), 0)
                return 0

            lax.fori_loop(0, NCH, ch_body, 0)
            return 0

        if k0 > 0:
            upsample(k0 - 1, (k0 - 1) % 3, 0)
        upsample(k0, k0 % 3, 0)

        def band_body(k, _):
            kp = lax.rem(k, 2)
            pltpu.make_async_copy(guide_src(k), gbuf.at[kp], gsem).wait()

            @pl.when(k < k0 + nb - 1)
            def _():
                pltpu.async_copy(guide_src(k + 1), gbuf.at[1 - kp], gsem)

            ynext = jnp.minimum(k + 1, Hg - 1)
            upsample(ynext, lax.rem(ynext, 3), 0)
            ya = jnp.maximum(k - 1, 0)
            sa = lax.rem(ya, 3)
            sb = lax.rem(k, 3)
            sc_ = lax.rem(ynext, 3)

            for h, (obuf, osem) in enumerate(((obuf0, osem0), (obuf1, osem1))):
                s0 = sa if h == 0 else sb
                s1 = sb if h == 0 else sc_
                s0base = s0 * (CD * CW)
                s1base = s1 * (CD * CW)

                @pl.when(k > k0)
                def _():
                    pltpu.make_async_copy(obuf, out_dst(k - 1, h), osem).wait()

                def row_body(rr, _):
                    wy = ((rr.astype(jnp.float32) + 0.5) * (1.0 / scale)
                          + (0.5 if h == 0 else 0.0))

                    def ch_body(j, _):
                        wl = j * 16 + lane
                        g = gbuf[kp, h * half + rr, pl.ds(j * 16, 16)]
                        gz = jnp.clip(g * D - 0.5, 0.0, D - 1.0)
                        iz0 = jnp.minimum(gz.astype(jnp.int32), D - 2)
                        wz = gz - iz0.astype(jnp.float32)
                        zoff = iz0 * CW + wl
                        f0 = zoff + s0base
                        f1 = zoff + s1base
                        loads = [(plsc.load_gather(rbuf, [f0 + (c * D * CW)]),
                                  plsc.load_gather(rbuf, [f1 + (c * D * CW)]),
                                  plsc.load_gather(rbuf, [f0 + (c * D * CW + CW)]),
                                  plsc.load_gather(rbuf, [f1 + (c * D * CW + CW)]))
                                 for c in range(C)]
                        for c, (a0, b0, a1, b1) in enumerate(loads):
                            r0 = a0 + wy * (b0 - a0)
                            r1 = a1 + wy * (b1 - a1)
                            obuf[c, rr, pl.ds(j * 16, 16)] = r0 + wz * (r1 - r0)
                        return 0

                    lax.fori_loop(0, NCH, ch_body, 0)
                    return 0

                lax.fori_loop(0, half, row_body, 0)
                pltpu.async_copy(obuf, out_dst(k, h), osem)
            return 0

        lax.fori_loop(k0, k0 + nb, band_body, 0)
        pltpu.make_async_copy(obuf0, out_dst(k0 + nb - 1, 0), osem0).wait()
        pltpu.make_async_copy(obuf1, out_dst(k0 + nb - 1, 1), osem1).wait()

    return sck(grid3, guidemap)


def kernel(bilateral_grid, guidemap):
    Bn, C, D, Hg, Wg = bilateral_grid.shape
    H, W = guidemap.shape[2], guidemap.shape[3]
    gridT = jnp.transpose(bilateral_grid, (0, 3, 1, 2, 4))

    if SC_ROWS == 0:
        return _tc_slice(gridT, guidemap, 0)
    grid3 = gridT.reshape(Bn, Hg * C * D * Wg)
    sc_part = _sc_slice(grid3, guidemap, SC_ROWS, Hg, C, D, Wg)
    if SC_ROWS == H:
        return sc_part
    out = _tc_slice(gridT, guidemap, SC_ROWS)
    return jax.lax.dynamic_update_slice(out, sc_part, (0, 0, 0, 0))
